# baseline (speedup 1.0000x reference)
import jax
import jax.numpy as jnp
from jax import lax
from jax.experimental import layout as jex_layout
from jax.experimental import pallas as pl
from jax.experimental.pallas import tpu as pltpu
from jax.experimental.pallas import tpu_sc as plsc

EMBED_DIM = 64

NC = 2
NS = 16
NW = NC * NS

K = 4


def _make_lookup(B, L):
    rows_per_w = B // NW
    n_groups = rows_per_w // K

    def body(idx_hbm, table_hbm, out_hbm,
             idx_v, buf_a, buf_b, sem_a, sem_b, osem_a, osem_b):
        wid = lax.axis_index("s") * NC + lax.axis_index("c")
        row0 = wid * rows_per_w

        pltpu.sync_copy(idx_hbm.at[pl.ds(row0, rows_per_w)], idx_v)

        def fire(g, buf, sem):
            for k in range(K):
                pltpu.make_async_copy(
                    table_hbm.at[idx_v.at[g * K + k]], buf.at[k], sem
                ).start()

        def drain(g, buf, sem):
            for k in range(K):
                pltpu.make_async_copy(
                    table_hbm.at[idx_v.at[g * K + k]], buf.at[k], sem
                ).wait()

        def out_copy(g, buf, osem):
            return pltpu.make_async_copy(
                buf, out_hbm.at[pl.ds(row0 + g * K, K)], osem)

        fire(0, buf_a, sem_a)
        fire(1, buf_b, sem_b)

        def step(g, buf, sem, osem):
            drain(g, buf, sem)
            out_copy(g, buf, osem).start()

            @pl.when(g + 2 < n_groups)
            def _():
                out_copy(g, buf, osem).wait()
                fire(g + 2, buf, sem)

        def loop(i, _):
            g = 2 * i
            step(g, buf_a, sem_a, osem_a)
            step(g + 1, buf_b, sem_b, osem_b)
            return 0

        lax.fori_loop(0, n_groups // 2, loop, 0)

        out_copy(n_groups - 2, buf_a, osem_a).wait()
        out_copy(n_groups - 1, buf_b, osem_b).wait()

    mesh = plsc.VectorSubcoreMesh(core_axis_name="c", subcore_axis_name="s")
    return pl.kernel(
        body,
        out_type=jax.ShapeDtypeStruct((B, L, EMBED_DIM), jnp.float32),
        mesh=mesh,
        compiler_params=pltpu.CompilerParams(use_tc_tiling_on_sc=False),
        scratch_types=[
            pltpu.VMEM((rows_per_w, L), jnp.int32),
            pltpu.VMEM((K, L, EMBED_DIM), jnp.float32),
            pltpu.VMEM((K, L, EMBED_DIM), jnp.float32),
            pltpu.SemaphoreType.DMA,
            pltpu.SemaphoreType.DMA,
            pltpu.SemaphoreType.DMA,
            pltpu.SemaphoreType.DMA,
        ],
    )


def _lookup_jit(batch, table):
    B, L = batch.shape
    assert B % (NW * K * 2) == 0
    return _make_lookup(B, L)(batch.astype(jnp.int32), table)


_jit_cache = {}


def kernel(batch, table):
    sharding = getattr(table, "sharding", None)
    fn = _jit_cache.get(sharding)
    if fn is None:
        if sharding is not None:
            fmt = jex_layout.Format(
                jex_layout.Layout(major_to_minor=(0, 1, 2)), sharding)
            fn = jax.jit(_lookup_jit, out_shardings=fmt)
        else:
            fn = jax.jit(_lookup_jit)
        _jit_cache[sharding] = fn
    return fn(batch, table)


# speedup vs baseline: 1.0020x; 1.0020x over previous
import jax
import jax.numpy as jnp
from jax import lax
from jax.experimental import pallas as pl
from jax.experimental.pallas import tpu as pltpu
from jax.experimental.pallas import tpu_sc as plsc

EMBED_DIM = 64

NC = 2
NS = 16
NW = NC * NS

K = 4


def _make_lookup(B, L):
    rows_per_w = B // NW
    n_groups = rows_per_w // K

    def body(idx_hbm, table_hbm, out_hbm,
             idx_v, buf_a, buf_b, sem_a, sem_b, osem_a, osem_b):
        wid = lax.axis_index("s") * NC + lax.axis_index("c")
        row0 = wid * rows_per_w

        pltpu.sync_copy(idx_hbm.at[pl.ds(row0, rows_per_w)], idx_v)

        def fire(g, buf, sem):
            for k in range(K):
                pltpu.make_async_copy(
                    table_hbm.at[idx_v.at[g * K + k]], buf.at[k], sem
                ).start()

        def drain(g, buf, sem):
            for k in range(K):
                pltpu.make_async_copy(
                    table_hbm.at[idx_v.at[g * K + k]], buf.at[k], sem
                ).wait()

        def out_copy(g, buf, osem):
            return pltpu.make_async_copy(
                buf, out_hbm.at[pl.ds(row0 + g * K, K)], osem)

        fire(0, buf_a, sem_a)
        fire(1, buf_b, sem_b)

        def step(g, buf, sem, osem):
            drain(g, buf, sem)
            out_copy(g, buf, osem).start()

            @pl.when(g + 2 < n_groups)
            def _():
                out_copy(g, buf, osem).wait()
                fire(g + 2, buf, sem)

        def loop(i, _):
            g = 2 * i
            step(g, buf_a, sem_a, osem_a)
            step(g + 1, buf_b, sem_b, osem_b)
            return 0

        lax.fori_loop(0, n_groups // 2, loop, 0)

        out_copy(n_groups - 2, buf_a, osem_a).wait()
        out_copy(n_groups - 1, buf_b, osem_b).wait()

    mesh = plsc.VectorSubcoreMesh(core_axis_name="c", subcore_axis_name="s")
    return pl.kernel(
        body,
        out_type=jax.ShapeDtypeStruct((B, L, EMBED_DIM), jnp.float32),
        mesh=mesh,
        compiler_params=pltpu.CompilerParams(use_tc_tiling_on_sc=False),
        scratch_types=[
            pltpu.VMEM((rows_per_w, L), jnp.int32),
            pltpu.VMEM((K, L, EMBED_DIM), jnp.float32),
            pltpu.VMEM((K, L, EMBED_DIM), jnp.float32),
            pltpu.SemaphoreType.DMA,
            pltpu.SemaphoreType.DMA,
            pltpu.SemaphoreType.DMA,
            pltpu.SemaphoreType.DMA,
        ],
    )


@jax.jit
def kernel(batch, table):
    B, L = batch.shape
    assert B % (NW * K * 2) == 0
    return _make_lookup(B, L)(batch.astype(jnp.int32), table)
